# baseline (speedup 1.0000x reference)
import functools

import jax
import jax.numpy as jnp
from jax import lax
from jax.experimental import pallas as pl
from jax.experimental.pallas import tpu as pltpu
from jax.experimental.pallas import tpu_sc as plsc

E = 64
D = 768
H = 768
OUT = 768
T = 2048
CAP = 40
NSLOT = E * CAP
NTRASH = 80
NSLOT_PAD = NSLOT + NTRASH
NC = 2
NS = 16
NW = NC * NS
TOK_PER_W = T // NW
SLOT_PER_W = NSLOT // NW
LANES = 16


def _gate_body(p_ref, wr_ref, wc_ref, sel_ref, slot_ref, aux_ref):
    probs = p_ref[...]
    wr = wr_ref[...]
    wc = wc_ref[...]
    sel = sel_ref[...]
    col = lax.broadcasted_iota(jnp.int32, (T, E), 1)
    onehot = (col == sel).astype(jnp.float32)

    r_iota = lax.broadcasted_iota(jnp.int32, (T, T), 0)
    c_iota = lax.broadcasted_iota(jnp.int32, (T, T), 1)
    prec = (wc > wr) | ((wc == wr) & (c_iota < r_iota))
    pmat = prec.astype(jnp.bfloat16)
    cmat = jnp.dot(pmat, onehot.astype(jnp.bfloat16),
                   preferred_element_type=jnp.float32)
    rank = jnp.sum(onehot * cmat, axis=-1, keepdims=True).astype(jnp.int32)

    tok = lax.broadcasted_iota(jnp.int32, (T, 1), 0)
    trash = NSLOT + (tok & (TOK_PER_W - 1))
    slot_ref[...] = jnp.where(rank < CAP, sel * CAP + rank, trash)

    counts = jnp.sum(onehot, axis=0, keepdims=True)
    importance = jnp.sum(probs, axis=0, keepdims=True)
    avg_prob = importance / T
    balance = E * jnp.sum(avg_prob * (counts / T))
    mean_imp = jnp.sum(importance) / E
    var = jnp.sum((importance - mean_imp) ** 2) / (E - 1)
    cv = jnp.sqrt(var) / (mean_imp + 1e-10)
    aux_ref[...] = jnp.full((1, 1), balance + 0.01 * cv, jnp.float32)


_gate = pl.pallas_call(
    _gate_body,
    out_shape=[
        jax.ShapeDtypeStruct((T, 1), jnp.int32),
        jax.ShapeDtypeStruct((1, 1), jnp.float32),
    ],
)


def _wid():
    return lax.axis_index("s") * NC + lax.axis_index("c")


def _sc_scatter_in_body(slot_hbm, flat_hbm, out_hbm, idx_v, rows_v, sem):
    base = _wid() * TOK_PER_W
    pltpu.sync_copy(slot_hbm.at[pl.ds(base, TOK_PER_W)], idx_v)
    pltpu.sync_copy(flat_hbm.at[pl.ds(base, TOK_PER_W)], rows_v)
    pltpu.async_copy(rows_v, out_hbm.at[idx_v], sem).wait()


def _sc_combine_body(slot_hbm, eout_hbm, flat_hbm, out_hbm,
                     idx_v, rows_v, res_v, sem):
    base = _wid() * TOK_PER_W
    pltpu.sync_copy(slot_hbm.at[pl.ds(base, TOK_PER_W)], idx_v)
    pltpu.sync_copy(flat_hbm.at[pl.ds(base, TOK_PER_W)], res_v)
    pltpu.async_copy(eout_hbm.at[idx_v], rows_v, sem).wait()

    def row_add(r, carry):
        for j in range(OUT // LANES):
            sl = pl.ds(j * LANES, LANES)
            res_v[r, sl] = res_v[r, sl] + rows_v[r, sl]
        return carry

    lax.fori_loop(0, TOK_PER_W, row_add, 0)
    pltpu.sync_copy(res_v, out_hbm.at[pl.ds(base, TOK_PER_W)])


@functools.lru_cache(maxsize=None)
def _sc_kernels():
    mesh = plsc.VectorSubcoreMesh(
        core_axis_name="c", subcore_axis_name="s", num_cores=NC)
    scatter_in = pl.kernel(
        _sc_scatter_in_body,
        mesh=mesh,
        out_type=jax.ShapeDtypeStruct((NSLOT_PAD, D), jnp.float32),
        scratch_types=[
            pltpu.VMEM((TOK_PER_W,), jnp.int32),
            pltpu.VMEM((TOK_PER_W, D), jnp.float32),
            pltpu.SemaphoreType.DMA,
        ],
    )
    combine = pl.kernel(
        _sc_combine_body,
        mesh=mesh,
        out_type=jax.ShapeDtypeStruct((T, OUT), jnp.float32),
        scratch_types=[
            pltpu.VMEM((TOK_PER_W,), jnp.int32),
            pltpu.VMEM((TOK_PER_W, OUT), jnp.float32),
            pltpu.VMEM((TOK_PER_W, OUT), jnp.float32),
            pltpu.SemaphoreType.DMA,
        ],
    )
    return scatter_in, combine


_HSPLIT = 2
_HCHUNK = H // _HSPLIT


def _ffn_body(x_ref, w1_ref, b1_ref, w2_ref, b2_ref, o_ref):
    e = pl.program_id(0)
    j = pl.program_id(1)

    @pl.when(e < E)
    def _():
        h = jnp.dot(x_ref[...], w1_ref[0], preferred_element_type=jnp.float32)
        h = jnp.maximum(h + b1_ref[0], 0.0)
        contrib = jnp.dot(h, w2_ref[0], preferred_element_type=jnp.float32)

        @pl.when(j == 0)
        def _():
            o_ref[...] = contrib + b2_ref[0]

        @pl.when(j > 0)
        def _():
            o_ref[...] = o_ref[...] + contrib

    @pl.when((e >= E) & (j == 0))
    def _():
        o_ref[...] = jnp.zeros((CAP, OUT), jnp.float32)


def _emap(e):
    return jnp.minimum(e, E - 1)


_ffn = pl.pallas_call(
    _ffn_body,
    grid=(NSLOT_PAD // CAP, _HSPLIT),
    in_specs=[
        pl.BlockSpec((CAP, D), lambda e, j: (_emap(e), 0)),
        pl.BlockSpec((1, D, _HCHUNK), lambda e, j: (_emap(e), 0, j)),
        pl.BlockSpec((1, 1, _HCHUNK), lambda e, j: (_emap(e), 0, j)),
        pl.BlockSpec((1, _HCHUNK, OUT), lambda e, j: (_emap(e), j, 0)),
        pl.BlockSpec((1, 1, OUT), lambda e, j: (_emap(e), 0, 0)),
    ],
    out_specs=pl.BlockSpec((CAP, OUT), lambda e, j: (e, 0)),
    out_shape=jax.ShapeDtypeStruct((NSLOT_PAD, OUT), jnp.float32),
)


def kernel(x, gate_W, gate_b, temperature, W1, b1, W2, b2):
    B, S, _ = x.shape
    flat = x.reshape(T, D)
    logits = flat @ gate_W + gate_b
    logits = logits / jnp.abs(temperature)
    probs = jax.nn.softmax(logits.astype(jnp.float32), axis=-1)
    w = jnp.max(probs, axis=-1, keepdims=True)
    sel = jnp.argmax(probs, axis=-1).astype(jnp.int32)[:, None]
    wn = w / jnp.sum(w, axis=-1, keepdims=True)
    slot2d, aux2d = _gate(probs, wn, wn.reshape(1, T), sel)
    slots = slot2d.reshape(T)
    sc_scatter_in, sc_combine = _sc_kernels()
    expert_in = sc_scatter_in(slots, flat)
    expert_out = _ffn(
        expert_in, W1, b1.reshape(E, 1, H), W2, b2.reshape(E, 1, OUT))
    out = sc_combine(slots, expert_out, flat)
    return out.reshape(B, S, OUT), aux2d.reshape(())


# speedup vs baseline: 1.2860x; 1.2860x over previous
import functools

import jax
import jax.numpy as jnp
from jax import lax
from jax.experimental import pallas as pl
from jax.experimental.pallas import tpu as pltpu
from jax.experimental.pallas import tpu_sc as plsc

E = 64
D = 768
H = 768
OUT = 768
T = 2048
CAP = 40
NSLOT = E * CAP
NTRASH = 80
NSLOT_PAD = NSLOT + NTRASH
NC = 2
NS = 16
NW = NC * NS
TOK_PER_W = T // NW
SLOT_PER_W = NSLOT // NW
LANES = 16


def _gate_body(p_ref, wr_ref, wc_ref, sel_ref, slot_ref, aux_ref):
    probs = p_ref[...]
    wr = wr_ref[...]
    wc = wc_ref[...]
    sel = sel_ref[...]
    col = lax.broadcasted_iota(jnp.int32, (T, E), 1)
    onehot = (col == sel).astype(jnp.float32)

    r_iota = lax.broadcasted_iota(jnp.int32, (T, T), 0)
    c_iota = lax.broadcasted_iota(jnp.int32, (T, T), 1)
    prec = (wc > wr) | ((wc == wr) & (c_iota < r_iota))
    pmat = prec.astype(jnp.bfloat16)
    cmat = jnp.dot(pmat, onehot.astype(jnp.bfloat16),
                   preferred_element_type=jnp.float32)
    rank = jnp.sum(onehot * cmat, axis=-1, keepdims=True).astype(jnp.int32)

    tok = lax.broadcasted_iota(jnp.int32, (T, 1), 0)
    trash = NSLOT + (tok & (TOK_PER_W - 1))
    slot_ref[...] = jnp.where(rank < CAP, sel * CAP + rank, trash)

    counts = jnp.sum(onehot, axis=0, keepdims=True)
    importance = jnp.sum(probs, axis=0, keepdims=True)
    avg_prob = importance / T
    balance = E * jnp.sum(avg_prob * (counts / T))
    mean_imp = jnp.sum(importance) / E
    var = jnp.sum((importance - mean_imp) ** 2) / (E - 1)
    cv = jnp.sqrt(var) / (mean_imp + 1e-10)
    aux_ref[...] = jnp.full((1, 1), balance + 0.01 * cv, jnp.float32)


_gate = pl.pallas_call(
    _gate_body,
    out_shape=[
        jax.ShapeDtypeStruct((T, 1), jnp.int32),
        jax.ShapeDtypeStruct((1, 1), jnp.float32),
    ],
)


def _wid():
    return lax.axis_index("s") * NC + lax.axis_index("c")


def _sc_scatter_in_body(slot_hbm, flat_hbm, out_hbm, idx_v, rows_v, sem):
    base = _wid() * TOK_PER_W
    pltpu.sync_copy(slot_hbm.at[pl.ds(base, TOK_PER_W)], idx_v)
    pltpu.sync_copy(flat_hbm.at[pl.ds(base, TOK_PER_W)], rows_v)
    pltpu.async_copy(rows_v, out_hbm.at[idx_v], sem).wait()


def _sc_combine_body(slot_hbm, eout_hbm, flat_hbm, out_hbm,
                     idx_v, rows_v, res_v, sem):
    base = _wid() * TOK_PER_W
    pltpu.sync_copy(slot_hbm.at[pl.ds(base, TOK_PER_W)], idx_v)
    pltpu.sync_copy(flat_hbm.at[pl.ds(base, TOK_PER_W)], res_v)
    pltpu.async_copy(eout_hbm.at[idx_v], rows_v, sem).wait()

    def row_add(r, carry):
        for j in range(OUT // LANES):
            sl = pl.ds(j * LANES, LANES)
            res_v[r, sl] = res_v[r, sl] + rows_v[r, sl]
        return carry

    lax.fori_loop(0, TOK_PER_W, row_add, 0)
    pltpu.sync_copy(res_v, out_hbm.at[0, pl.ds(base, TOK_PER_W)])


@functools.lru_cache(maxsize=None)
def _sc_kernels():
    mesh = plsc.VectorSubcoreMesh(
        core_axis_name="c", subcore_axis_name="s", num_cores=NC)
    scatter_in = pl.kernel(
        _sc_scatter_in_body,
        mesh=mesh,
        out_type=jax.ShapeDtypeStruct((NSLOT_PAD, D), jnp.float32),
        scratch_types=[
            pltpu.VMEM((TOK_PER_W,), jnp.int32),
            pltpu.VMEM((TOK_PER_W, D), jnp.float32),
            pltpu.SemaphoreType.DMA,
        ],
    )
    combine = pl.kernel(
        _sc_combine_body,
        mesh=mesh,
        out_type=jax.ShapeDtypeStruct((1, T, OUT), jnp.float32),
        scratch_types=[
            pltpu.VMEM((TOK_PER_W,), jnp.int32),
            pltpu.VMEM((TOK_PER_W, OUT), jnp.float32),
            pltpu.VMEM((TOK_PER_W, OUT), jnp.float32),
            pltpu.SemaphoreType.DMA,
        ],
    )
    return scatter_in, combine


def _ffn_body(x_ref, w1_ref, b1_ref, w2_ref, b2_ref, o_ref):
    e = pl.program_id(0)

    @pl.when(e < E)
    def _():
        h = jnp.dot(x_ref[...], w1_ref[0], preferred_element_type=jnp.float32)
        h = jnp.maximum(h + b1_ref[0], 0.0)
        o_ref[...] = (
            jnp.dot(h, w2_ref[0], preferred_element_type=jnp.float32)
            + b2_ref[0])

    @pl.when(e >= E)
    def _():
        o_ref[...] = jnp.zeros((CAP, OUT), jnp.float32)


def _emap(e):
    return jnp.minimum(e, E - 1)


_ffn = pl.pallas_call(
    _ffn_body,
    grid=(NSLOT_PAD // CAP,),
    in_specs=[
        pl.BlockSpec((CAP, D), lambda e: (_emap(e), 0)),
        pl.BlockSpec((1, D, H), lambda e: (_emap(e), 0, 0)),
        pl.BlockSpec((1, 1, H), lambda e: (_emap(e), 0, 0)),
        pl.BlockSpec((1, H, OUT), lambda e: (_emap(e), 0, 0)),
        pl.BlockSpec((1, 1, OUT), lambda e: (_emap(e), 0, 0)),
    ],
    out_specs=pl.BlockSpec((CAP, OUT), lambda e: (e, 0)),
    out_shape=jax.ShapeDtypeStruct((NSLOT_PAD, OUT), jnp.float32),
)


def kernel(x, gate_W, gate_b, temperature, W1, b1, W2, b2):
    B, S, _ = x.shape
    flat = x.reshape(T, D)
    logits = flat @ gate_W + gate_b
    logits = logits / jnp.abs(temperature)
    probs = jax.nn.softmax(logits.astype(jnp.float32), axis=-1)
    w = jnp.max(probs, axis=-1, keepdims=True)
    sel = jnp.argmax(probs, axis=-1).astype(jnp.int32)[:, None]
    wn = w / jnp.sum(w, axis=-1, keepdims=True)
    slot2d, aux2d = _gate(probs, wn, wn.reshape(1, T), sel)
    slots = slot2d.reshape(T)
    sc_scatter_in, sc_combine = _sc_kernels()
    expert_in = sc_scatter_in(slots, flat)
    expert_out = _ffn(
        expert_in, W1, b1.reshape(E, 1, H), W2, b2.reshape(E, 1, OUT))
    out = sc_combine(slots, expert_out, flat)
    return out, aux2d.reshape(())
